# baseline (speedup 1.0000x reference)
import functools

import jax
import jax.numpy as jnp
from jax import lax
from jax.experimental import pallas as pl
from jax.experimental.pallas import tpu as pltpu
from jax.experimental.pallas import tpu_sc as plsc

_N_USERS = 100000
_N_ITEMS = 100000
_D = 64
_L = 50
_B = 4096

_NC = 2
_NS = 16
_NW = _NC * _NS

def _sc_mesh():
    return plsc.VectorSubcoreMesh(core_axis_name="c", subcore_axis_name="s",
                                  num_cores=_NC, num_subcores=_NS)


def _sc_hist_gather(hist_pack, user_idx, item_idx):
    n_per_w = _B // _NW

    @functools.partial(
        pl.kernel,
        mesh=_sc_mesh(),
        out_type=(
            jax.ShapeDtypeStruct((_B, 128), jnp.int32),
            jax.ShapeDtypeStruct((_B, 128), jnp.int32),
        ),
        scratch_types=[
            pltpu.VMEM((n_per_w,), jnp.int32),
            pltpu.VMEM((n_per_w, 128), jnp.int32),
            pltpu.SemaphoreType.DMA,
        ],
    )
    def k(h_hbm, ui_hbm, ii_hbm, ou_hbm, oi_hbm, idx_v, rows_v, sem):
        wid = lax.axis_index("s") * _NC + lax.axis_index("c")
        base = wid * n_per_w
        pltpu.sync_copy(ui_hbm.at[pl.ds(base, n_per_w)], idx_v)
        pltpu.async_copy(h_hbm.at[idx_v], rows_v, sem).wait()
        pltpu.sync_copy(rows_v, ou_hbm.at[pl.ds(base, n_per_w)])
        pltpu.sync_copy(ii_hbm.at[pl.ds(base, n_per_w)], idx_v)
        pltpu.async_copy(h_hbm.at[idx_v], rows_v, sem).wait()
        pltpu.sync_copy(rows_v, oi_hbm.at[pl.ds(base, n_per_w)])

    return k(hist_pack, user_idx, item_idx)


def _sc_embed_gather(u_pack, i_pack, flat_u, flat_i, user_idx, item_idx, nb):
    n_big = nb * _L
    big_per_w = n_big // _NW
    chunk = 400
    n_chunks = big_per_w // chunk
    t_per_w = nb // _NW

    @functools.partial(
        pl.kernel,
        mesh=_sc_mesh(),
        out_type=(
            jax.ShapeDtypeStruct((n_big, 128), jnp.float32),
            jax.ShapeDtypeStruct((n_big, 128), jnp.float32),
            jax.ShapeDtypeStruct((nb, 128), jnp.float32),
            jax.ShapeDtypeStruct((nb, 128), jnp.float32),
        ),
        scratch_types=[
            pltpu.VMEM((t_per_w,), jnp.int32),
            pltpu.VMEM((t_per_w, 128), jnp.float32),
            pltpu.VMEM((chunk,), jnp.int32),
            pltpu.VMEM((chunk, 128), jnp.float32),
            pltpu.VMEM((chunk,), jnp.int32),
            pltpu.VMEM((chunk, 128), jnp.float32),
            pltpu.SemaphoreType.DMA,
            pltpu.SemaphoreType.DMA,
            pltpu.SemaphoreType.DMA,
            pltpu.SemaphoreType.DMA,
            pltpu.SemaphoreType.DMA,
        ],
    )
    def k(up_hbm, ip_hbm, fu_hbm, fi_hbm, ui_hbm, ii_hbm,
          gu_hbm, gi_hbm, ut_hbm, it_hbm,
          idx_t, rows_t, idx_b0, rows_b0, idx_b1, rows_b1,
          sem_t, sem_g0, sem_g1, sem_w0, sem_w1):
        wid = lax.axis_index("s") * _NC + lax.axis_index("c")

        tbase = wid * t_per_w
        pltpu.sync_copy(ui_hbm.at[pl.ds(tbase, t_per_w)], idx_t)
        pltpu.async_copy(up_hbm.at[idx_t], rows_t, sem_t).wait()
        pltpu.sync_copy(rows_t, ut_hbm.at[pl.ds(tbase, t_per_w)])
        pltpu.sync_copy(ii_hbm.at[pl.ds(tbase, t_per_w)], idx_t)
        pltpu.async_copy(ip_hbm.at[idx_t], rows_t, sem_t).wait()
        pltpu.sync_copy(rows_t, it_hbm.at[pl.ds(tbase, t_per_w)])

        work = ([(j, fu_hbm, ip_hbm, gu_hbm) for j in range(n_chunks)]
                + [(j, fi_hbm, up_hbm, gi_hbm) for j in range(n_chunks)])
        bufs = [(idx_b0, rows_b0, sem_g0, sem_w0),
                (idx_b1, rows_b1, sem_g1, sem_w1)]
        pend_w = [None, None]
        pend_g = [None, None]

        def start_gather(step):
            j, f_hbm, pack_hbm, _ = work[step]
            ib, rb, sg, _sw = bufs[step % 2]
            if pend_w[step % 2] is not None:
                pend_w[step % 2].wait()
            base = wid * big_per_w + j * chunk
            pltpu.sync_copy(f_hbm.at[pl.ds(base, chunk)], ib)
            pend_g[step % 2] = pltpu.async_copy(pack_hbm.at[ib], rb, sg)

        start_gather(0)
        for step in range(len(work)):
            if step + 1 < len(work):
                start_gather(step + 1)
            j, _, _, out_hbm = work[step]
            ib, rb, _sg, sw = bufs[step % 2]
            pend_g[step % 2].wait()
            base = wid * big_per_w + j * chunk
            pend_w[step % 2] = pltpu.async_copy(
                rb, out_hbm.at[pl.ds(base, chunk)], sw)
        pend_w[0].wait()
        pend_w[1].wait()

    return k(u_pack, i_pack, flat_u, flat_i, user_idx, item_idx)


def _tc_pack_embed(uet, ueh, iet, ieh):
    rows = 100352
    blk = 1024
    grid = (rows // blk,)

    def body(a_ref, b_ref, c_ref, d_ref, u_ref, i_ref):
        u_ref[...] = jnp.concatenate([a_ref[...], b_ref[...]], axis=1)
        i_ref[...] = jnp.concatenate([c_ref[...], d_ref[...]], axis=1)

    return pl.pallas_call(
        body,
        grid=grid,
        in_specs=[pl.BlockSpec((blk, _D), lambda i: (i, 0))] * 4,
        out_specs=[pl.BlockSpec((blk, 128), lambda i: (i, 0))] * 2,
        out_shape=[jax.ShapeDtypeStruct((rows, 128), jnp.float32)] * 2,
        compiler_params=pltpu.CompilerParams(
            dimension_semantics=("parallel",),
        ),
    )(uet, ueh, iet, ieh)


def _tc_pack_hist(user_hist, item_hist):
    rows = 100000
    blk = 1000
    grid = (rows // blk,)

    def body(a_ref, b_ref, o_ref):
        z = jnp.zeros((a_ref.shape[0], 64 - _L), jnp.int32)
        o_ref[...] = jnp.concatenate([a_ref[...], z, b_ref[...], z], axis=1)

    return pl.pallas_call(
        body,
        grid=grid,
        in_specs=[pl.BlockSpec((blk, _L), lambda i: (i, 0))] * 2,
        out_specs=pl.BlockSpec((blk, 128), lambda i: (i, 0)),
        out_shape=jax.ShapeDtypeStruct((rows, 128), jnp.int32),
        compiler_params=pltpu.CompilerParams(
            dimension_semantics=("parallel",),
        ),
    )(user_hist, item_hist)


def _attn_body(gu_ref, gi_ref, idxu_ref, idxi_ref, uix_ref, iix_ref,
               wut_ref, bu_ref, gu_glob_ref, wit_ref, bi_ref, gi_glob_ref,
               hu_ref, hi_ref):
    def one_side(r_ref, idx_ref, tgt_ref, wt_ref, b_ref, g_ref, pad_id, o_ref):
        R = r_ref[...][:, :, _D:]
        bblk = R.shape[0]
        idx = idx_ref[...][:, :, None]
        tgt = tgt_ref[...][:, :, None]
        K = jnp.tanh(
            jnp.dot(R.reshape(bblk * _L, _D).astype(jnp.bfloat16), wt_ref[...],
                    preferred_element_type=jnp.float32)
            + b_ref[...]
        ).reshape(bblk, _L, _D)
        g = g_ref[...].reshape(1, 1, _D)
        s = jnp.sum(K * g, axis=-1, keepdims=True) * (1.0 / 8.0)
        pad_mask = idx == pad_id
        mask = pad_mask | (idx == tgt)
        s = jnp.where(mask, -1e9, s)
        m = jnp.max(s, axis=1, keepdims=True)
        e = jnp.exp(s - m)
        w = e / jnp.sum(e, axis=1, keepdims=True)
        w = jnp.where(pad_mask, 0.0, w)
        o_ref[...] = jnp.sum(R * w, axis=1)

    one_side(gu_ref, idxu_ref, iix_ref, wut_ref, bu_ref, gu_glob_ref,
             _N_ITEMS, hu_ref)
    one_side(gi_ref, idxi_ref, uix_ref, wit_ref, bi_ref, gi_glob_ref,
             _N_USERS, hi_ref)


def _tc_attention(g_u, g_i, idx_u, idx_i, user_idx2, item_idx2,
                  wut, bu, gu_glob, wit, bi, gi_glob):
    nb = g_u.shape[0]
    bblk = 128
    grid = (nb // bblk,)
    full = lambda i: (0, 0)
    return pl.pallas_call(
        _attn_body,
        grid=grid,
        in_specs=[
            pl.BlockSpec((bblk, _L, 128), lambda i: (i, 0, 0)),
            pl.BlockSpec((bblk, _L, 128), lambda i: (i, 0, 0)),
            pl.BlockSpec((bblk, _L), lambda i: (i, 0)),
            pl.BlockSpec((bblk, _L), lambda i: (i, 0)),
            pl.BlockSpec((bblk, 1), lambda i: (i, 0)),
            pl.BlockSpec((bblk, 1), lambda i: (i, 0)),
            pl.BlockSpec((_D, _D), full),
            pl.BlockSpec((1, _D), full),
            pl.BlockSpec((1, _D), full),
            pl.BlockSpec((_D, _D), full),
            pl.BlockSpec((1, _D), full),
            pl.BlockSpec((1, _D), full),
        ],
        out_specs=[
            pl.BlockSpec((bblk, _D), lambda i: (i, 0)),
            pl.BlockSpec((bblk, _D), lambda i: (i, 0)),
        ],
        out_shape=[
            jax.ShapeDtypeStruct((nb, _D), jnp.float32),
            jax.ShapeDtypeStruct((nb, _D), jnp.float32),
        ],
        compiler_params=pltpu.CompilerParams(
            dimension_semantics=("parallel",),
        ),
    )(g_u, g_i, idx_u, idx_i, user_idx2, item_idx2,
      wut, bu, gu_glob, wit, bi, gi_glob)


def kernel(user_embed_target_W, item_embed_target_W, user_embed_hist_W,
           item_embed_hist_W, user_embed_global, item_embed_global,
           proj_u_W, proj_u_b, proj_i_W, proj_i_b,
           user_idx, item_idx, user_hist, item_hist):
    hist_pack = _tc_pack_hist(user_hist, item_hist)
    u_pack, i_pack = _tc_pack_embed(
        user_embed_target_W, user_embed_hist_W,
        item_embed_target_W, item_embed_hist_W)

    ref_u_pad, ref_i_pad = _sc_hist_gather(hist_pack, user_idx, item_idx)

    idx_u = ref_u_pad[:, :_L]
    idx_i = ref_i_pad[:, 64:64 + _L]

    wut = proj_u_W.T.astype(jnp.bfloat16)
    wit = proj_i_W.T.astype(jnp.bfloat16)
    bu = proj_u_b.reshape(1, _D)
    bi = proj_i_b.reshape(1, _D)
    gu_glob = user_embed_global.reshape(1, _D)
    gi_glob = item_embed_global.reshape(1, _D)

    nslc = 2
    nb = _B // nslc
    u_ts, i_ts, hus, his = [], [], [], []
    for s in range(nslc):
        sl = slice(s * nb, (s + 1) * nb)
        g_u, g_i, u_t128, i_t128 = _sc_embed_gather(
            u_pack, i_pack,
            idx_u[sl].reshape(-1), idx_i[sl].reshape(-1),
            user_idx[sl], item_idx[sl], nb)
        hu, hi = _tc_attention(
            g_u.reshape(nb, _L, 128), g_i.reshape(nb, _L, 128),
            idx_u[sl], idx_i[sl],
            user_idx[sl].reshape(nb, 1), item_idx[sl].reshape(nb, 1),
            wut, bu, gu_glob, wit, bi, gi_glob)
        u_ts.append(u_t128[:, :_D])
        i_ts.append(i_t128[:, :_D])
        hus.append(hu)
        his.append(hi)

    u_t = jnp.concatenate(u_ts, axis=0)
    i_t = jnp.concatenate(i_ts, axis=0)
    hu = jnp.concatenate(hus, axis=0)
    hi = jnp.concatenate(his, axis=0)

    id_cat = jnp.concatenate([u_t, i_t], axis=-1)
    hist_cat = jnp.concatenate([hu, hi], axis=-1)
    user_cat = jnp.concatenate([u_t, hu], axis=-1)
    item_cat = jnp.concatenate([i_t, hi], axis=-1)
    return (id_cat, hist_cat, user_cat, item_cat)


# speedup vs baseline: 1.5275x; 1.5275x over previous
import functools

import jax
import jax.numpy as jnp
from jax import lax
from jax.experimental import pallas as pl
from jax.experimental.pallas import tpu as pltpu
from jax.experimental.pallas import tpu_sc as plsc

_N_USERS = 100000
_N_ITEMS = 100000
_D = 64
_L = 50
_B = 4096

_NC = 2
_NS = 16
_NW = _NC * _NS

def _sc_mesh():
    return plsc.VectorSubcoreMesh(core_axis_name="c", subcore_axis_name="s",
                                  num_cores=_NC, num_subcores=_NS)


def _sc_hist_gather(hist_pack, user_idx, item_idx):
    n_per_w = _B // _NW

    @functools.partial(
        pl.kernel,
        mesh=_sc_mesh(),
        out_type=(
            jax.ShapeDtypeStruct((_B, 128), jnp.int32),
            jax.ShapeDtypeStruct((_B, 128), jnp.int32),
        ),
        scratch_types=[
            pltpu.VMEM((n_per_w,), jnp.int32),
            pltpu.VMEM((n_per_w, 128), jnp.int32),
            pltpu.SemaphoreType.DMA,
        ],
    )
    def k(h_hbm, ui_hbm, ii_hbm, ou_hbm, oi_hbm, idx_v, rows_v, sem):
        wid = lax.axis_index("s") * _NC + lax.axis_index("c")
        base = wid * n_per_w
        pltpu.sync_copy(ui_hbm.at[pl.ds(base, n_per_w)], idx_v)
        pltpu.async_copy(h_hbm.at[idx_v], rows_v, sem).wait()
        pltpu.sync_copy(rows_v, ou_hbm.at[pl.ds(base, n_per_w)])
        pltpu.sync_copy(ii_hbm.at[pl.ds(base, n_per_w)], idx_v)
        pltpu.async_copy(h_hbm.at[idx_v], rows_v, sem).wait()
        pltpu.sync_copy(rows_v, oi_hbm.at[pl.ds(base, n_per_w)])

    return k(hist_pack, user_idx, item_idx)


def _sc_embed_gather(pack_t, pack_bu, pack_bi, flat_u, flat_i,
                     user_idx, item_idx, nb):
    n_big = nb * _L
    big_per_w = n_big // _NW
    chunk = 400
    n_chunks = big_per_w // chunk
    t_per_w = nb // _NW

    @functools.partial(
        pl.kernel,
        mesh=_sc_mesh(),
        out_type=(
            jax.ShapeDtypeStruct((n_big, 128), jnp.float32),
            jax.ShapeDtypeStruct((n_big, 128), jnp.float32),
            jax.ShapeDtypeStruct((nb, 128), jnp.float32),
            jax.ShapeDtypeStruct((nb, 128), jnp.float32),
        ),
        scratch_types=[
            pltpu.VMEM((t_per_w,), jnp.int32),
            pltpu.VMEM((t_per_w, 128), jnp.float32),
            pltpu.VMEM((chunk,), jnp.int32),
            pltpu.VMEM((chunk, 128), jnp.float32),
            pltpu.VMEM((chunk,), jnp.int32),
            pltpu.VMEM((chunk, 128), jnp.float32),
            pltpu.SemaphoreType.DMA,
            pltpu.SemaphoreType.DMA,
            pltpu.SemaphoreType.DMA,
            pltpu.SemaphoreType.DMA,
            pltpu.SemaphoreType.DMA,
        ],
    )
    def k(pt_hbm, pu_hbm, pi_hbm, fu_hbm, fi_hbm, ui_hbm, ii_hbm,
          gu_hbm, gi_hbm, ut_hbm, it_hbm,
          idx_t, rows_t, idx_b0, rows_b0, idx_b1, rows_b1,
          sem_t, sem_g0, sem_g1, sem_w0, sem_w1):
        wid = lax.axis_index("s") * _NC + lax.axis_index("c")

        tbase = wid * t_per_w
        pltpu.sync_copy(ui_hbm.at[pl.ds(tbase, t_per_w)], idx_t)
        pltpu.async_copy(pt_hbm.at[idx_t], rows_t, sem_t).wait()
        pltpu.sync_copy(rows_t, ut_hbm.at[pl.ds(tbase, t_per_w)])
        pltpu.sync_copy(ii_hbm.at[pl.ds(tbase, t_per_w)], idx_t)
        pltpu.async_copy(pt_hbm.at[idx_t], rows_t, sem_t).wait()
        pltpu.sync_copy(rows_t, it_hbm.at[pl.ds(tbase, t_per_w)])

        work = ([(j, fu_hbm, pi_hbm, gu_hbm) for j in range(n_chunks)]
                + [(j, fi_hbm, pu_hbm, gi_hbm) for j in range(n_chunks)])
        bufs = [(idx_b0, rows_b0, sem_g0, sem_w0),
                (idx_b1, rows_b1, sem_g1, sem_w1)]
        pend_w = [None, None]
        pend_g = [None, None]

        def start_gather(step):
            j, f_hbm, pack_hbm, _ = work[step]
            ib, rb, sg, _sw = bufs[step % 2]
            if pend_w[step % 2] is not None:
                pend_w[step % 2].wait()
            base = wid * big_per_w + j * chunk
            pltpu.sync_copy(f_hbm.at[pl.ds(base, chunk)], ib)
            pend_g[step % 2] = pltpu.async_copy(pack_hbm.at[ib], rb, sg)

        start_gather(0)
        for step in range(len(work)):
            if step + 1 < len(work):
                start_gather(step + 1)
            j, _, _, out_hbm = work[step]
            ib, rb, _sg, sw = bufs[step % 2]
            pend_g[step % 2].wait()
            base = wid * big_per_w + j * chunk
            pend_w[step % 2] = pltpu.async_copy(
                rb, out_hbm.at[pl.ds(base, chunk)], sw)
        pend_w[0].wait()
        pend_w[1].wait()

    return k(pack_t, pack_bu, pack_bi, flat_u, flat_i, user_idx, item_idx)


def _tc_pack_embed(uet, ueh, iet, ieh):
    rows = 100352
    blk = 1024
    grid = (rows // blk,)

    def body(a_ref, b_ref, c_ref, d_ref, t_ref, u_ref, i_ref):
        i_blk = pl.program_id(0)
        rowid = (jax.lax.broadcasted_iota(jnp.int32, (blk, _D), 0)
                 + blk * i_blk).astype(jnp.float32)
        idhalf = jnp.where(rowid == 100000.0, -1e9, rowid)
        t_ref[...] = jnp.concatenate([a_ref[...], c_ref[...]], axis=1)
        u_ref[...] = jnp.concatenate([idhalf, b_ref[...]], axis=1)
        i_ref[...] = jnp.concatenate([idhalf, d_ref[...]], axis=1)

    return pl.pallas_call(
        body,
        grid=grid,
        in_specs=[pl.BlockSpec((blk, _D), lambda i: (i, 0))] * 4,
        out_specs=[pl.BlockSpec((blk, 128), lambda i: (i, 0))] * 3,
        out_shape=[jax.ShapeDtypeStruct((rows, 128), jnp.float32)] * 3,
        compiler_params=pltpu.CompilerParams(
            dimension_semantics=("parallel",),
        ),
    )(uet, ueh, iet, ieh)


def _tc_pack_hist(user_hist, item_hist):
    rows = 100000
    blk = 1000
    grid = (rows // blk,)

    def body(a_ref, b_ref, o_ref):
        z = jnp.zeros((a_ref.shape[0], 64 - _L), jnp.int32)
        o_ref[...] = jnp.concatenate([a_ref[...], z, b_ref[...], z], axis=1)

    return pl.pallas_call(
        body,
        grid=grid,
        in_specs=[pl.BlockSpec((blk, _L), lambda i: (i, 0))] * 2,
        out_specs=pl.BlockSpec((blk, 128), lambda i: (i, 0)),
        out_shape=jax.ShapeDtypeStruct((rows, 128), jnp.int32),
        compiler_params=pltpu.CompilerParams(
            dimension_semantics=("parallel",),
        ),
    )(user_hist, item_hist)


def _attn_body(gu_ref, gi_ref, uix_ref, iix_ref,
               wut_ref, bu_ref, gbu_ref, wit_ref, bi_ref, gbi_ref,
               hu_ref, hi_ref):
    def one_side(r_ref, tgt_ref, wt_ref, b_ref, gb_ref, o_ref):
        Rp = r_ref[...]
        bblk = Rp.shape[1]
        Rf = Rp.reshape(_L * bblk, 128)
        K = jnp.tanh(
            jnp.dot(Rf[:, _D:].astype(jnp.bfloat16), wt_ref[...],
                    preferred_element_type=jnp.float32)
            + b_ref[...])
        S = jnp.dot(K.astype(jnp.bfloat16), gb_ref[...],
                    preferred_element_type=jnp.float32)
        S3 = S.reshape(_L, bblk, _D)
        idden = Rp[:, :, :_D]
        tgt = tgt_ref[...].reshape(1, bblk, 1)
        pen = jnp.where(idden == tgt, -1e9, jnp.minimum(idden, 0.0))
        e = jnp.exp(S3 + pen)
        sm = jnp.sum(e, axis=0, keepdims=True)
        w = e * (1.0 / sm)
        o_ref[...] = jnp.sum(w * Rp[:, :, _D:], axis=0)

    one_side(gu_ref, iix_ref, wut_ref, bu_ref, gbu_ref, hu_ref)
    one_side(gi_ref, uix_ref, wit_ref, bi_ref, gbi_ref, hi_ref)


def _tc_attention(g_u, g_i, user_idx_f, item_idx_f,
                  wut, bu, gbu, wit, bi, gbi):
    nb = g_u.shape[1]
    bblk = 256
    grid = (nb // bblk,)
    full = lambda i: (0, 0)
    return pl.pallas_call(
        _attn_body,
        grid=grid,
        in_specs=[
            pl.BlockSpec((_L, bblk, 128), lambda i: (0, i, 0)),
            pl.BlockSpec((_L, bblk, 128), lambda i: (0, i, 0)),
            pl.BlockSpec((1, bblk), lambda i: (0, i)),
            pl.BlockSpec((1, bblk), lambda i: (0, i)),
            pl.BlockSpec((_D, _D), full),
            pl.BlockSpec((1, _D), full),
            pl.BlockSpec((_D, _D), full),
            pl.BlockSpec((_D, _D), full),
            pl.BlockSpec((1, _D), full),
            pl.BlockSpec((_D, _D), full),
        ],
        out_specs=[
            pl.BlockSpec((bblk, _D), lambda i: (i, 0)),
            pl.BlockSpec((bblk, _D), lambda i: (i, 0)),
        ],
        out_shape=[
            jax.ShapeDtypeStruct((nb, _D), jnp.float32),
            jax.ShapeDtypeStruct((nb, _D), jnp.float32),
        ],
        compiler_params=pltpu.CompilerParams(
            dimension_semantics=("parallel",),
        ),
    )(g_u, g_i, user_idx_f, item_idx_f,
      wut, bu, gbu, wit, bi, gbi)


def kernel(user_embed_target_W, item_embed_target_W, user_embed_hist_W,
           item_embed_hist_W, user_embed_global, item_embed_global,
           proj_u_W, proj_u_b, proj_i_W, proj_i_b,
           user_idx, item_idx, user_hist, item_hist):
    hist_pack = _tc_pack_hist(user_hist, item_hist)
    pack_t, pack_bu, pack_bi = _tc_pack_embed(
        user_embed_target_W, user_embed_hist_W,
        item_embed_target_W, item_embed_hist_W)

    ref_u_pad, ref_i_pad = _sc_hist_gather(hist_pack, user_idx, item_idx)

    idx_u = ref_u_pad[:, :_L]
    idx_i = ref_i_pad[:, 64:64 + _L]

    wut = proj_u_W.T.astype(jnp.bfloat16)
    wit = proj_i_W.T.astype(jnp.bfloat16)
    bu = proj_u_b.reshape(1, _D)
    bi = proj_i_b.reshape(1, _D)
    gbu = jnp.broadcast_to((user_embed_global * 0.125)[:, None],
                           (_D, _D)).astype(jnp.bfloat16)
    gbi = jnp.broadcast_to((item_embed_global * 0.125)[:, None],
                           (_D, _D)).astype(jnp.bfloat16)
    uixf = user_idx.astype(jnp.float32)
    iixf = item_idx.astype(jnp.float32)

    nslc = 2
    nb = _B // nslc
    u_ts, i_ts, hus, his = [], [], [], []
    for s in range(nslc):
        sl = slice(s * nb, (s + 1) * nb)
        g_u, g_i, u_t128, i_t128 = _sc_embed_gather(
            pack_t, pack_bu, pack_bi,
            idx_u[sl].T.reshape(-1), idx_i[sl].T.reshape(-1),
            user_idx[sl], item_idx[sl], nb)
        hu, hi = _tc_attention(
            g_u.reshape(_L, nb, 128), g_i.reshape(_L, nb, 128),
            uixf[sl].reshape(1, nb), iixf[sl].reshape(1, nb),
            wut, bu, gbu, wit, bi, gbi)
        u_ts.append(u_t128[:, :_D])
        i_ts.append(i_t128[:, _D:])
        hus.append(hu)
        his.append(hi)

    u_t = jnp.concatenate(u_ts, axis=0)
    i_t = jnp.concatenate(i_ts, axis=0)
    hu = jnp.concatenate(hus, axis=0)
    hi = jnp.concatenate(his, axis=0)

    id_cat = jnp.concatenate([u_t, i_t], axis=-1)
    hist_cat = jnp.concatenate([hu, hi], axis=-1)
    user_cat = jnp.concatenate([u_t, hu], axis=-1)
    item_cat = jnp.concatenate([i_t, hi], axis=-1)
    return (id_cat, hist_cat, user_cat, item_cat)
